# Initial kernel scaffold; baseline (speedup 1.0000x reference)
import jax
import jax.numpy as jnp
from jax.experimental import pallas as pl


def kernel(mid_his_input, cat_his_input, mid_table, cat_table):
    raise NotImplementedError("write your pallas kernel here")


# speedup vs baseline: 6.6634x; 6.6634x over previous
import functools

import jax
import jax.numpy as jnp
from jax import lax
from jax.experimental import pallas as pl
from jax.experimental.pallas import tpu as pltpu
from jax.experimental.pallas import tpu_sc as plsc

N_MID = 1000000
N_CAT = 100000
EMBED_DIM = 32
BATCH = 16384
MAX_LEN = 200

ROWS = BATCH * MAX_LEN
NW = 32
G = 128
CH = 4
ROWS_PER_W = ROWS // NW
GROUPS_PER_W = ROWS_PER_W // G
CHUNKS_PER_W = GROUPS_PER_W // CH


def _body(mid_idx_hbm, cat_idx_hbm, mid_table, cat_table, out_hbm,
          midx_v, cidx_v, mrows_v, crows_v, sem):
    wid = lax.axis_index("c") * 16 + lax.axis_index("s")
    gbase0 = wid * GROUPS_PER_W

    def chunk(t, _):
        gbase = gbase0 + t * CH
        rbase = gbase * G
        pltpu.sync_copy(mid_idx_hbm.at[pl.ds(gbase, CH)], midx_v)
        pltpu.sync_copy(cat_idx_hbm.at[pl.ds(gbase, CH)], cidx_v)
        copies = []
        for j in range(CH):
            cm = pltpu.make_async_copy(
                mid_table.at[midx_v.at[j]], mrows_v.at[pl.ds(j * G, G)], sem)
            cc = pltpu.make_async_copy(
                cat_table.at[cidx_v.at[j]], crows_v.at[pl.ds(j * G, G)], sem)
            cm.start()
            cc.start()
            copies.append(cm)
            copies.append(cc)
        for c in copies:
            c.wait()
        pltpu.sync_copy(mrows_v, out_hbm.at[pl.ds(rbase, CH * G), 0])
        pltpu.sync_copy(crows_v, out_hbm.at[pl.ds(rbase, CH * G), 1])
        return ()

    lax.fori_loop(0, CHUNKS_PER_W, chunk, (), unroll=False)


@jax.jit
def _run(mid_idx, cat_idx, mid_table, cat_table):
    mesh = plsc.VectorSubcoreMesh(core_axis_name="c", subcore_axis_name="s")
    f = pl.kernel(
        _body,
        out_type=jax.ShapeDtypeStruct((ROWS, 2, EMBED_DIM), jnp.float32),
        mesh=mesh,
        scratch_types=[
            pltpu.VMEM((CH, G), jnp.int32),
            pltpu.VMEM((CH, G), jnp.int32),
            pltpu.VMEM((CH * G, EMBED_DIM), jnp.float32),
            pltpu.VMEM((CH * G, EMBED_DIM), jnp.float32),
            pltpu.SemaphoreType.DMA,
        ],
        compiler_params=pltpu.CompilerParams(use_tc_tiling_on_sc=False),
    )
    return f(mid_idx, cat_idx, mid_table, cat_table)


def kernel(mid_his_input, cat_his_input, mid_table, cat_table):
    mid_idx = mid_his_input.reshape(ROWS // G, G).astype(jnp.int32)
    cat_idx = cat_his_input.reshape(ROWS // G, G).astype(jnp.int32)
    out = _run(mid_idx, cat_idx, mid_table, cat_table)
    return out.reshape(BATCH, MAX_LEN, 2 * EMBED_DIM)


# speedup vs baseline: 6.9327x; 1.0404x over previous
import jax
import jax.numpy as jnp
from jax import lax
from jax.experimental import pallas as pl
from jax.experimental.pallas import tpu as pltpu
from jax.experimental.pallas import tpu_sc as plsc

N_MID = 1000000
N_CAT = 100000
EMBED_DIM = 32
BATCH = 16384
MAX_LEN = 200

ROWS = BATCH * MAX_LEN
NW = 32
BPC = 4
BATCH_PER_W = BATCH // NW
CHUNKS_PER_W = BATCH_PER_W // BPC
GATHER_SPLITS = ((0, 128), (128, 72))


def _body(mid_idx_hbm, cat_idx_hbm, mid_table, cat_table, out_hbm,
          midx_v, cidx_v, mrows_v, crows_v, sem):
    wid = lax.axis_index("c") * 16 + lax.axis_index("s")
    b0 = wid * BATCH_PER_W

    def chunk(t, _):
        b = b0 + t * BPC
        pltpu.sync_copy(mid_idx_hbm.at[pl.ds(b * MAX_LEN, BPC * MAX_LEN)],
                        midx_v)
        pltpu.sync_copy(cat_idx_hbm.at[pl.ds(b * MAX_LEN, BPC * MAX_LEN)],
                        cidx_v)
        copies = []
        for i in range(BPC):
            for (l0, n) in GATHER_SPLITS:
                cm = pltpu.make_async_copy(
                    mid_table.at[midx_v.at[pl.ds(i * MAX_LEN + l0, n)]],
                    mrows_v.at[i, pl.ds(l0, n), :], sem)
                cc = pltpu.make_async_copy(
                    cat_table.at[cidx_v.at[pl.ds(i * MAX_LEN + l0, n)]],
                    crows_v.at[i, pl.ds(l0, n), :], sem)
                cm.start()
                cc.start()
                copies.append(cm)
                copies.append(cc)
        for c in copies:
            c.wait()
        pltpu.sync_copy(mrows_v,
                        out_hbm.at[pl.ds(b, BPC), :, pl.ds(0, EMBED_DIM)])
        pltpu.sync_copy(crows_v,
                        out_hbm.at[pl.ds(b, BPC), :,
                                   pl.ds(EMBED_DIM, EMBED_DIM)])
        return ()

    lax.fori_loop(0, CHUNKS_PER_W, chunk, (), unroll=False)


@jax.jit
def _run(mid_idx, cat_idx, mid_table, cat_table):
    mesh = plsc.VectorSubcoreMesh(core_axis_name="c", subcore_axis_name="s")
    f = pl.kernel(
        _body,
        out_type=jax.ShapeDtypeStruct((BATCH, MAX_LEN, 2 * EMBED_DIM),
                                      jnp.float32),
        mesh=mesh,
        scratch_types=[
            pltpu.VMEM((BPC * MAX_LEN,), jnp.int32),
            pltpu.VMEM((BPC * MAX_LEN,), jnp.int32),
            pltpu.VMEM((BPC, MAX_LEN, EMBED_DIM), jnp.float32),
            pltpu.VMEM((BPC, MAX_LEN, EMBED_DIM), jnp.float32),
            pltpu.SemaphoreType.DMA,
        ],
        compiler_params=pltpu.CompilerParams(use_tc_tiling_on_sc=False),
    )
    return f(mid_idx, cat_idx, mid_table, cat_table)


def kernel(mid_his_input, cat_his_input, mid_table, cat_table):
    mid_idx = mid_his_input.reshape(ROWS).astype(jnp.int32)
    cat_idx = cat_his_input.reshape(ROWS).astype(jnp.int32)
    return _run(mid_idx, cat_idx, mid_table, cat_table)
